# baseline (speedup 1.0000x reference)
import functools

import jax
import jax.numpy as jnp
from jax import lax
from jax.experimental import pallas as pl
from jax.experimental.pallas import tpu as pltpu
from jax.experimental.pallas import tpu_sc as plsc

_B, _F, _V, _D = 16384, 26, 100000, 50
_DP = 64
_NL = (_F * _DP) // 128
_EPS = 1e-5
_NC, _NS = 2, 16
_NW = _NC * _NS
_BF = _B * _F
_PER_W = _BF // _NW
_CH = 128
_NCH = _PER_W // _CH
_TW = _F * _V * 2


def _gather_body(table_hbm, idxg_hbm, idxs_hbm, out_hbm,
                 idxg_v, idxs_v, win_a, win_b, sem_a, sem_b, sem_oa, sem_ob):
    wid = lax.axis_index("s") * _NC + lax.axis_index("c")
    pltpu.sync_copy(idxg_hbm.at[pl.ds(wid * _NCH, _NCH)], idxg_v)
    pltpu.sync_copy(idxs_hbm.at[pl.ds(wid * _NCH, _NCH)], idxs_v)

    def fire(j, win, sem):
        pltpu.async_copy(table_hbm.at[idxg_v.at[j]], win, sem)

    def drain(win, sem):
        pltpu.make_async_copy(table_hbm.at[idxg_v.at[0]], win, sem).wait()

    def scat_wait(win, sem_o):
        pltpu.make_async_copy(win, out_hbm.at[idxs_v.at[0]], sem_o).wait()

    fire(0, win_a, sem_a)

    def body(j, _):
        even = lax.rem(j, 2) == 0

        @pl.when(jnp.logical_and(j + 1 < _NCH, even))
        def _():
            @pl.when(j >= 1)
            def _():
                scat_wait(win_b, sem_ob)
            fire(j + 1, win_b, sem_b)

        @pl.when(jnp.logical_and(j + 1 < _NCH, jnp.logical_not(even)))
        def _():
            scat_wait(win_a, sem_oa)
            fire(j + 1, win_a, sem_a)

        @pl.when(even)
        def _():
            drain(win_a, sem_a)
            pltpu.async_copy(win_a, out_hbm.at[idxs_v.at[j]], sem_oa)

        @pl.when(jnp.logical_not(even))
        def _():
            drain(win_b, sem_b)
            pltpu.async_copy(win_b, out_hbm.at[idxs_v.at[j]], sem_ob)

        return 0

    lax.fori_loop(0, _NCH, body, 0)
    scat_wait(win_a, sem_oa)
    scat_wait(win_b, sem_ob)


@functools.cache
def _sc_gather():
    return pl.kernel(
        _gather_body,
        out_type=jax.ShapeDtypeStruct((_BF, _DP), jnp.float32),
        mesh=plsc.VectorSubcoreMesh(
            core_axis_name="c", subcore_axis_name="s",
            num_cores=_NC, num_subcores=_NS),
        scratch_types=[
            pltpu.VMEM((_NCH, _CH), jnp.int32),
            pltpu.VMEM((_NCH, _CH), jnp.int32),
            pltpu.VMEM((_CH, _DP), jnp.float32),
            pltpu.VMEM((_CH, _DP), jnp.float32),
            pltpu.SemaphoreType.DMA,
            pltpu.SemaphoreType.DMA,
            pltpu.SemaphoreType.DMA,
            pltpu.SemaphoreType.DMA,
        ],
        compiler_params=pltpu.CompilerParams(
            use_tc_tiling_on_sc=False, needs_layout_passes=False),
    )


_BB = 512


def _mlp1_body(x_ref, num_ref, w1e_ref, w1b_ref, b1_ref, h1_ref, stats_ref,
               acc_ref, hacc_ref):
    i = pl.program_id(0)
    c = pl.program_id(1)

    @pl.when(jnp.logical_and(i == 0, c == 0))
    def _():
        acc_ref[...] = jnp.zeros_like(acc_ref)

    @pl.when(c == 0)
    def _():
        hacc_ref[...] = jnp.dot(num_ref[...], w1b_ref[...],
                                preferred_element_type=jnp.float32)

    hacc_ref[...] += jnp.dot(x_ref[0], w1e_ref[0],
                             preferred_element_type=jnp.float32)

    @pl.when(c == _NL - 1)
    def _():
        h = jnp.maximum(hacc_ref[...] + b1_ref[...], 0.0)
        h1_ref[...] = h
        acc_ref[0:1, :] += jnp.sum(h, axis=0, keepdims=True)
        acc_ref[1:2, :] += jnp.sum(h * h, axis=0, keepdims=True)

        @pl.when(i == pl.num_programs(0) - 1)
        def _():
            stats_ref[...] = acc_ref[...]


def _mlp2_body(h1_ref, stats_ref, g1_ref, be1_ref, w2_ref, b2_ref, g2_ref,
               be2_ref, w3r_ref, b3_ref, out_ref):
    s = stats_ref[...]
    m1 = s[0:1, :] * (1.0 / _B)
    v1 = s[1:2, :] * (1.0 / _B) - m1 * m1
    h1n = (h1_ref[...] - m1) * lax.rsqrt(v1 + _EPS) * g1_ref[...] + be1_ref[...]
    h2 = jnp.dot(h1n, w2_ref[...], preferred_element_type=jnp.float32)
    h2 = jnp.maximum(h2 + b2_ref[...], 0.0)
    m2 = jnp.mean(h2, axis=0, keepdims=True)
    v2 = jnp.mean(h2 * h2, axis=0, keepdims=True) - m2 * m2
    h2n = (h2 - m2) * lax.rsqrt(v2 + _EPS) * g2_ref[...] + be2_ref[...]
    out_ref[...] = jnp.sum(h2n * w3r_ref[...], axis=1, keepdims=True) + b3_ref[...]


def _mlp1(x3, nump, w1e3, w1b, b1r):
    grid = (_B // _BB, _NL)
    return pl.pallas_call(
        _mlp1_body,
        grid=grid,
        in_specs=[
            pl.BlockSpec((1, _BB, 128), lambda i, c: (c, i, 0)),
            pl.BlockSpec((_BB, 16), lambda i, c: (i, 0)),
            pl.BlockSpec((1, 128, 128), lambda i, c: (c, 0, 0)),
            pl.BlockSpec((16, 128), lambda i, c: (0, 0)),
            pl.BlockSpec((1, 128), lambda i, c: (0, 0)),
        ],
        out_specs=[
            pl.BlockSpec((_BB, 128), lambda i, c: (i, 0)),
            pl.BlockSpec((8, 128), lambda i, c: (0, 0)),
        ],
        out_shape=[
            jax.ShapeDtypeStruct((_B, 128), jnp.float32),
            jax.ShapeDtypeStruct((8, 128), jnp.float32),
        ],
        scratch_shapes=[pltpu.VMEM((8, 128), jnp.float32),
                        pltpu.VMEM((_BB, 128), jnp.float32)],
    )(x3, nump, w1e3, w1b, b1r)


def _mlp2(h1, stats, g1r, be1r, W2, b2r, g2r, be2r, w3r, b3r):
    return pl.pallas_call(
        _mlp2_body,
        out_shape=jax.ShapeDtypeStruct((_B, 1), jnp.float32),
    )(h1, stats, g1r, be1r, W2, b2r, g2r, be2r, w3r, b3r)


def kernel(cat_data, num_data, emb_tables, W1, b1, g1, be1, W2, b2, g2, be2, W3, b3):
    t64 = jnp.pad(emb_tables, ((0, 0), (0, 0), (0, 128 - _D))).reshape(_TW, _DP)

    offs = (jnp.arange(_F, dtype=jnp.int32) * _V)[None, :]
    flat_idx = (cat_data + offs).reshape(_BF)
    idxg = (flat_idx * 2).reshape(_NW * _NCH, _CH)

    bb = (jnp.arange(_B, dtype=jnp.int32) * 2)[:, None]
    ff = jnp.arange(_F, dtype=jnp.int32)[None, :]
    idxs = ((ff >> 1) * (2 * _B) + bb + (ff & 1)).reshape(_NW * _NCH, _CH)

    x64 = _sc_gather()(t64, idxg, idxs)
    x3 = x64.reshape(_NL, _B, 128)

    nump = jnp.pad(num_data, ((0, 0), (0, 3)))
    w1e3 = jnp.pad(W1[:_F * _D].reshape(_F, _D, 128),
                   ((0, 0), (0, _DP - _D), (0, 0))).reshape(_NL, 128, 128)
    w1b = jnp.pad(W1[_F * _D:], ((0, 3), (0, 0)))

    h1, stats = _mlp1(x3, nump, w1e3, w1b, b1.reshape(1, -1))
    out = _mlp2(h1, stats, g1.reshape(1, -1), be1.reshape(1, -1),
                W2, b2.reshape(1, -1), g2.reshape(1, -1), be2.reshape(1, -1),
                W3.reshape(1, -1), b3.reshape(1, 1))
    return out.reshape(_B)


# speedup vs baseline: 1.0968x; 1.0968x over previous
import functools

import jax
import jax.numpy as jnp
from jax import lax
from jax.experimental import pallas as pl
from jax.experimental.pallas import tpu as pltpu
from jax.experimental.pallas import tpu_sc as plsc

_B, _F, _V, _D = 16384, 26, 100000, 50
_DP = 64
_NL = (_F * _DP) // 128
_EPS = 1e-5
_NC, _NS = 2, 16
_NW = _NC * _NS
_BF = _B * _F
_PER_W = _BF // _NW
_CH = 128
_NCH = _PER_W // _CH
_TW = _F * _V * 2


def _gather_body(table_hbm, idxg_hbm, idxs_hbm, out_hbm,
                 idxg_v, idxs_v, win_a, win_b, sem_a, sem_b, sem_oa, sem_ob):
    wid = lax.axis_index("s") * _NC + lax.axis_index("c")
    pltpu.sync_copy(idxg_hbm.at[pl.ds(wid * _NCH, _NCH)], idxg_v)
    pltpu.sync_copy(idxs_hbm.at[pl.ds(wid * _NCH, _NCH)], idxs_v)

    def fire(j, win, sem):
        pltpu.async_copy(table_hbm.at[idxg_v.at[j]], win, sem)

    def drain(win, sem):
        pltpu.make_async_copy(table_hbm.at[idxg_v.at[0]], win, sem).wait()

    def scat_wait(win, sem_o):
        pltpu.make_async_copy(win, out_hbm.at[idxs_v.at[0]], sem_o).wait()

    fire(0, win_a, sem_a)

    def body(j, _):
        even = lax.rem(j, 2) == 0

        @pl.when(jnp.logical_and(j + 1 < _NCH, even))
        def _():
            @pl.when(j >= 1)
            def _():
                scat_wait(win_b, sem_ob)
            fire(j + 1, win_b, sem_b)

        @pl.when(jnp.logical_and(j + 1 < _NCH, jnp.logical_not(even)))
        def _():
            scat_wait(win_a, sem_oa)
            fire(j + 1, win_a, sem_a)

        @pl.when(even)
        def _():
            drain(win_a, sem_a)
            pltpu.async_copy(win_a, out_hbm.at[idxs_v.at[j]], sem_oa)

        @pl.when(jnp.logical_not(even))
        def _():
            drain(win_b, sem_b)
            pltpu.async_copy(win_b, out_hbm.at[idxs_v.at[j]], sem_ob)

        return 0

    lax.fori_loop(0, _NCH, body, 0)
    scat_wait(win_a, sem_oa)
    scat_wait(win_b, sem_ob)


@functools.cache
def _sc_gather():
    return pl.kernel(
        _gather_body,
        out_type=jax.ShapeDtypeStruct((_BF, _DP), jnp.float32),
        mesh=plsc.VectorSubcoreMesh(
            core_axis_name="c", subcore_axis_name="s",
            num_cores=_NC, num_subcores=_NS),
        scratch_types=[
            pltpu.VMEM((_NCH, _CH), jnp.int32),
            pltpu.VMEM((_NCH, _CH), jnp.int32),
            pltpu.VMEM((_CH, _DP), jnp.float32),
            pltpu.VMEM((_CH, _DP), jnp.float32),
            pltpu.SemaphoreType.DMA,
            pltpu.SemaphoreType.DMA,
            pltpu.SemaphoreType.DMA,
            pltpu.SemaphoreType.DMA,
        ],
        compiler_params=pltpu.CompilerParams(
            use_tc_tiling_on_sc=False, needs_layout_passes=False),
    )


_BB = 512


def _mlp1_body(x_ref, num_ref, w1e_ref, w1b_ref, b1_ref, h1_ref, stats_ref,
               acc_ref):
    i = pl.program_id(0)

    @pl.when(i == 0)
    def _():
        acc_ref[...] = jnp.zeros_like(acc_ref)

    h = jnp.dot(num_ref[...], w1b_ref[...], preferred_element_type=jnp.float32)
    for c in range(_NL):
        h = h + jnp.dot(x_ref[c], w1e_ref[c], preferred_element_type=jnp.float32)
    h = jnp.maximum(h + b1_ref[...], 0.0)
    h1_ref[...] = h
    acc_ref[0:1, :] += jnp.sum(h, axis=0, keepdims=True)
    acc_ref[1:2, :] += jnp.sum(h * h, axis=0, keepdims=True)

    @pl.when(i == pl.num_programs(0) - 1)
    def _():
        stats_ref[...] = acc_ref[...]


def _mlp2a_body(h1_ref, stats_ref, g1_ref, be1_ref, w2_ref, b2_ref,
                h2_ref, stats2_ref, acc_ref):
    i = pl.program_id(0)

    @pl.when(i == 0)
    def _():
        acc_ref[...] = jnp.zeros_like(acc_ref)

    s = stats_ref[...]
    m1 = s[0:1, :] * (1.0 / _B)
    v1 = s[1:2, :] * (1.0 / _B) - m1 * m1
    h1n = (h1_ref[...] - m1) * lax.rsqrt(v1 + _EPS) * g1_ref[...] + be1_ref[...]
    h2 = jnp.dot(h1n, w2_ref[...], preferred_element_type=jnp.float32)
    h2 = jnp.maximum(h2 + b2_ref[...], 0.0)
    h2_ref[...] = h2
    acc_ref[0:1, :] += jnp.sum(h2, axis=0, keepdims=True)
    acc_ref[1:2, :] += jnp.sum(h2 * h2, axis=0, keepdims=True)

    @pl.when(i == pl.num_programs(0) - 1)
    def _():
        stats2_ref[...] = acc_ref[...]


def _mlp2b_body(h2_ref, stats2_ref, g2_ref, be2_ref, w3r_ref, b3_ref, out_ref):
    s = stats2_ref[...]
    m2 = s[0:1, :] * (1.0 / _B)
    v2 = s[1:2, :] * (1.0 / _B) - m2 * m2
    h2n = (h2_ref[...] - m2) * lax.rsqrt(v2 + _EPS) * g2_ref[...] + be2_ref[...]
    out_ref[...] = jnp.sum(h2n * w3r_ref[...], axis=1, keepdims=True) + b3_ref[...]


def _mlp1(x3, nump, w1e3, w1b, b1r):
    grid = (_B // _BB,)
    return pl.pallas_call(
        _mlp1_body,
        grid=grid,
        in_specs=[
            pl.BlockSpec((_NL, _BB, 128), lambda i: (0, i, 0)),
            pl.BlockSpec((_BB, 16), lambda i: (i, 0)),
            pl.BlockSpec((_NL, 128, 128), lambda i: (0, 0, 0)),
            pl.BlockSpec((16, 128), lambda i: (0, 0)),
            pl.BlockSpec((1, 128), lambda i: (0, 0)),
        ],
        out_specs=[
            pl.BlockSpec((_BB, 128), lambda i: (i, 0)),
            pl.BlockSpec((8, 128), lambda i: (0, 0)),
        ],
        out_shape=[
            jax.ShapeDtypeStruct((_B, 128), jnp.float32),
            jax.ShapeDtypeStruct((8, 128), jnp.float32),
        ],
        scratch_shapes=[pltpu.VMEM((8, 128), jnp.float32)],
    )(x3, nump, w1e3, w1b, b1r)


def _mlp2(h1, stats, g1r, be1r, W2, b2r, g2r, be2r, w3r, b3r):
    grid = (_B // _BB,)
    h2, stats2 = pl.pallas_call(
        _mlp2a_body,
        grid=grid,
        in_specs=[
            pl.BlockSpec((_BB, 128), lambda i: (i, 0)),
            pl.BlockSpec((8, 128), lambda i: (0, 0)),
            pl.BlockSpec((1, 128), lambda i: (0, 0)),
            pl.BlockSpec((1, 128), lambda i: (0, 0)),
            pl.BlockSpec((128, 64), lambda i: (0, 0)),
            pl.BlockSpec((1, 64), lambda i: (0, 0)),
        ],
        out_specs=[
            pl.BlockSpec((_BB, 64), lambda i: (i, 0)),
            pl.BlockSpec((8, 64), lambda i: (0, 0)),
        ],
        out_shape=[
            jax.ShapeDtypeStruct((_B, 64), jnp.float32),
            jax.ShapeDtypeStruct((8, 64), jnp.float32),
        ],
        scratch_shapes=[pltpu.VMEM((8, 64), jnp.float32)],
    )(h1, stats, g1r, be1r, W2, b2r)
    return pl.pallas_call(
        _mlp2b_body,
        grid=grid,
        in_specs=[
            pl.BlockSpec((_BB, 64), lambda i: (i, 0)),
            pl.BlockSpec((8, 64), lambda i: (0, 0)),
            pl.BlockSpec((1, 64), lambda i: (0, 0)),
            pl.BlockSpec((1, 64), lambda i: (0, 0)),
            pl.BlockSpec((1, 64), lambda i: (0, 0)),
            pl.BlockSpec((1, 1), lambda i: (0, 0)),
        ],
        out_specs=pl.BlockSpec((_BB, 1), lambda i: (i, 0)),
        out_shape=jax.ShapeDtypeStruct((_B, 1), jnp.float32),
    )(h2, stats2, g2r, be2r, w3r, b3r)


def kernel(cat_data, num_data, emb_tables, W1, b1, g1, be1, W2, b2, g2, be2, W3, b3):
    t64 = jnp.pad(emb_tables, ((0, 0), (0, 0), (0, 128 - _D))).reshape(_TW, _DP)

    offs = (jnp.arange(_F, dtype=jnp.int32) * _V)[None, :]
    flat_idx = (cat_data + offs).reshape(_BF)
    idxg = (flat_idx * 2).reshape(_NW * _NCH, _CH)

    bb = (jnp.arange(_B, dtype=jnp.int32) * 2)[:, None]
    ff = jnp.arange(_F, dtype=jnp.int32)[None, :]
    idxs = ((ff >> 1) * (2 * _B) + bb + (ff & 1)).reshape(_NW * _NCH, _CH)

    x64 = _sc_gather()(t64, idxg, idxs)
    x3 = x64.reshape(_NL, _B, 128)

    nump = jnp.pad(num_data, ((0, 0), (0, 3)))
    w1e3 = jnp.pad(W1[:_F * _D].reshape(_F, _D, 128),
                   ((0, 0), (0, _DP - _D), (0, 0))).reshape(_NL, 128, 128)
    w1b = jnp.pad(W1[_F * _D:], ((0, 3), (0, 0)))

    h1, stats = _mlp1(x3, nump, w1e3, w1b, b1.reshape(1, -1))
    out = _mlp2(h1, stats, g1.reshape(1, -1), be1.reshape(1, -1),
                W2, b2.reshape(1, -1), g2.reshape(1, -1), be2.reshape(1, -1),
                W3.reshape(1, -1), b3.reshape(1, 1))
    return out.reshape(_B)
